# baseline (device time: 36588 ns/iter reference)
import jax
import jax.numpy as jnp
from jax import lax
from jax.experimental import pallas as pl
from jax.experimental.pallas import tpu as pltpu

N_DEV = 4
B, SQ, SKV = 2, 256, 256
HQ_LOC, DH = 4, 64
DM = 512
BLK = 64


def kernel(x, Wq, K_ext, V_ext, Wo):
    p = lax.axis_index("i")
    K_loc = lax.dynamic_slice_in_dim(K_ext, p * HQ_LOC, HQ_LOC, axis=2)
    V_loc = lax.dynamic_slice_in_dim(V_ext, p * HQ_LOC, HQ_LOC, axis=2)
    K_loc = jnp.transpose(K_loc, (0, 2, 1, 3))
    V_loc = jnp.transpose(V_loc, (0, 2, 1, 3))

    def body(x_ref, wq_ref, k_ref, v_ref, wo_ref, out_ref,
             recv_ref, send_sems, recv_sems):
        my_p = lax.axis_index("i")
        partner1 = my_p ^ 1
        partner2 = 3 - my_p

        barrier = pltpu.get_barrier_semaphore()
        for nbr in (partner1, partner2):
            pl.semaphore_signal(
                barrier, inc=1,
                device_id=(nbr,), device_id_type=pl.DeviceIdType.MESH,
            )
        pl.semaphore_wait(barrier, 2)

        qb = lax.broadcasted_iota(jnp.int32, (SQ, SKV), 0) // BLK
        kb = lax.broadcasted_iota(jnp.int32, (SQ, SKV), 1) // BLK
        mask = kb <= qb

        for b in range(B):
            q_all = jnp.dot(x_ref[b], wq_ref[...],
                            preferred_element_type=jnp.float32)
            acc = jnp.zeros((SQ, DM), jnp.float32)
            for h in range(HQ_LOC):
                q = q_all[:, h * DH:(h + 1) * DH]
                k = k_ref[b, h]
                v = v_ref[b, h]
                s = lax.dot_general(
                    q, k, (((1,), (1,)), ((), ())),
                    preferred_element_type=jnp.float32) * 0.125
                s = jnp.where(mask, s, -1e9)
                m = jnp.max(s, axis=-1, keepdims=True)
                w = jnp.exp(s - m)
                w = w / jnp.sum(w, axis=-1, keepdims=True)
                ctx = jnp.dot(w, v, preferred_element_type=jnp.float32)
                acc = acc + jnp.dot(ctx, wo_ref[h * DH:(h + 1) * DH, :],
                                    preferred_element_type=jnp.float32)
            out_ref[b] = acc

        for stage in range(2):
            partner = partner1 if stage == 0 else partner2
            rdma = pltpu.make_async_remote_copy(
                src_ref=out_ref,
                dst_ref=recv_ref.at[stage],
                send_sem=send_sems.at[stage],
                recv_sem=recv_sems.at[stage],
                device_id=(partner,),
                device_id_type=pl.DeviceIdType.MESH,
            )
            rdma.start()
            rdma.wait()
            out_ref[...] = out_ref[...] + recv_ref[stage]

    return pl.pallas_call(
        body,
        out_shape=jax.ShapeDtypeStruct((B, SQ, DM), jnp.float32),
        in_specs=[pl.BlockSpec(memory_space=pltpu.VMEM)] * 5,
        out_specs=pl.BlockSpec(memory_space=pltpu.VMEM),
        scratch_shapes=[
            pltpu.VMEM((2, B, SQ, DM), jnp.float32),
            pltpu.SemaphoreType.DMA((2,)),
            pltpu.SemaphoreType.DMA((2,)),
        ],
        compiler_params=pltpu.CompilerParams(collective_id=0),
    )(x, Wq, K_loc, V_loc, Wo)


# device time: 25424 ns/iter; 1.4391x vs baseline; 1.4391x over previous
import jax
import jax.numpy as jnp
from jax import lax
from jax.experimental import pallas as pl
from jax.experimental.pallas import tpu as pltpu

N_DEV = 4
B, SQ, SKV = 2, 256, 256
HQ_LOC, DH = 4, 64
DM = 512
BLK = 64


def kernel(x, Wq, K_ext, V_ext, Wo):
    p = lax.axis_index("i")
    K_loc = lax.dynamic_slice_in_dim(K_ext, p * HQ_LOC, HQ_LOC, axis=2)
    V_loc = lax.dynamic_slice_in_dim(V_ext, p * HQ_LOC, HQ_LOC, axis=2)
    K_loc = jnp.transpose(K_loc, (0, 2, 1, 3))
    V_loc = jnp.transpose(V_loc, (0, 2, 1, 3))

    def body(x_ref, wq_ref, k_ref, v_ref, wo_ref, out_ref,
             recv_ref, send_sems, recv_sems):
        my_p = lax.axis_index("i")
        partner1 = my_p ^ 1
        partner2 = 3 - my_p

        barrier = pltpu.get_barrier_semaphore()
        for nbr in (partner1, partner2):
            pl.semaphore_signal(
                barrier, inc=1,
                device_id=(nbr,), device_id_type=pl.DeviceIdType.MESH,
            )
        pl.semaphore_wait(barrier, 2)

        qb = lax.broadcasted_iota(jnp.int32, (SQ, SKV), 0) // BLK
        kb = lax.broadcasted_iota(jnp.int32, (SQ, SKV), 1) // BLK
        mask = kb <= qb

        def compute_partial(b):
            q_all = jnp.dot(x_ref[b], wq_ref[...],
                            preferred_element_type=jnp.float32)
            acc = jnp.zeros((SQ, DM), jnp.float32)
            for h in range(HQ_LOC):
                q = q_all[:, h * DH:(h + 1) * DH]
                k = k_ref[b, h]
                v = v_ref[b, h]
                s = lax.dot_general(
                    q, k, (((1,), (1,)), ((), ())),
                    preferred_element_type=jnp.float32) * 0.125
                s = jnp.where(mask, s, -1e9)
                m = jnp.max(s, axis=-1, keepdims=True)
                w = jnp.exp(s - m)
                w = w / jnp.sum(w, axis=-1, keepdims=True)
                ctx = jnp.dot(w, v, preferred_element_type=jnp.float32)
                acc = acc + jnp.dot(ctx, wo_ref[h * DH:(h + 1) * DH, :],
                                    preferred_element_type=jnp.float32)
            out_ref[b] = acc

        def exchange(stage, b, partner):
            idx = stage * 2 + b
            return pltpu.make_async_remote_copy(
                src_ref=out_ref.at[b],
                dst_ref=recv_ref.at[idx],
                send_sem=send_sems.at[idx],
                recv_sem=recv_sems.at[idx],
                device_id=(partner,),
                device_id_type=pl.DeviceIdType.MESH,
            )

        compute_partial(0)
        s0b0 = exchange(0, 0, partner1)
        s0b0.start()
        compute_partial(1)
        s0b1 = exchange(0, 1, partner2)
        s0b1.start()

        s0b0.wait()
        out_ref[0] = out_ref[0] + recv_ref[0]
        s1b0 = exchange(1, 0, partner2)
        s1b0.start()

        s0b1.wait()
        out_ref[1] = out_ref[1] + recv_ref[1]
        s1b1 = exchange(1, 1, partner1)
        s1b1.start()

        s1b0.wait()
        out_ref[0] = out_ref[0] + recv_ref[2]
        s1b1.wait()
        out_ref[1] = out_ref[1] + recv_ref[3]

    return pl.pallas_call(
        body,
        out_shape=jax.ShapeDtypeStruct((B, SQ, DM), jnp.float32),
        in_specs=[pl.BlockSpec(memory_space=pltpu.VMEM)] * 5,
        out_specs=pl.BlockSpec(memory_space=pltpu.VMEM),
        scratch_shapes=[
            pltpu.VMEM((4, SQ, DM), jnp.float32),
            pltpu.SemaphoreType.DMA((4,)),
            pltpu.SemaphoreType.DMA((4,)),
        ],
        compiler_params=pltpu.CompilerParams(collective_id=0),
    )(x, Wq, K_loc, V_loc, Wo)


# device time: 23952 ns/iter; 1.5276x vs baseline; 1.0615x over previous
import jax
import jax.numpy as jnp
from jax import lax
from jax.experimental import pallas as pl
from jax.experimental.pallas import tpu as pltpu

N_DEV = 4
B, SQ, SKV = 2, 256, 256
HQ_LOC, DH = 4, 64
DM = 512
BLK = 64


def kernel(x, Wq, K_ext, V_ext, Wo):
    p = lax.axis_index("i")
    K_loc = lax.dynamic_slice_in_dim(K_ext, p * HQ_LOC, HQ_LOC, axis=2)
    V_loc = lax.dynamic_slice_in_dim(V_ext, p * HQ_LOC, HQ_LOC, axis=2)
    K_loc = jnp.transpose(K_loc, (0, 2, 1, 3))
    V_loc = jnp.transpose(V_loc, (0, 2, 1, 3))

    def body(x_ref, wq_ref, k_ref, v_ref, wo_ref, out_ref,
             recv_ref, send_sems, recv_sems):
        my_p = lax.axis_index("i")
        partner1 = my_p ^ 1
        partner2 = 3 - my_p

        barrier = pltpu.get_barrier_semaphore()
        for nbr in (partner1, partner2):
            pl.semaphore_signal(
                barrier, inc=1,
                device_id=(nbr,), device_id_type=pl.DeviceIdType.MESH,
            )

        RC = SQ // 2

        def compute_chunk(b, r):
            kn = (r + 1) * RC
            rows = slice(r * RC, (r + 1) * RC)
            qb = lax.broadcasted_iota(jnp.int32, (RC, kn), 0) // BLK + (
                r * RC // BLK)
            kb = lax.broadcasted_iota(jnp.int32, (RC, kn), 1) // BLK
            mask = kb <= qb
            q_all = jnp.dot(x_ref[b, rows], wq_ref[...],
                            preferred_element_type=jnp.float32)
            acc = jnp.zeros((RC, DM), jnp.float32)
            for h in range(HQ_LOC):
                q = q_all[:, h * DH:(h + 1) * DH]
                k = k_ref[b, h, :kn]
                v = v_ref[b, h, :kn]
                s = lax.dot_general(
                    q, k, (((1,), (1,)), ((), ())),
                    preferred_element_type=jnp.float32) * 0.125
                s = jnp.where(mask, s, -1e9)
                m = jnp.max(s, axis=-1, keepdims=True)
                w = jnp.exp(s - m)
                w = w / jnp.sum(w, axis=-1, keepdims=True)
                ctx = jnp.dot(w, v, preferred_element_type=jnp.float32)
                acc = acc + jnp.dot(ctx, wo_ref[h * DH:(h + 1) * DH, :],
                                    preferred_element_type=jnp.float32)
            out_ref[b, rows] = acc

        def exchange(stage, b, r, partner):
            idx = stage * 4 + b * 2 + r
            return pltpu.make_async_remote_copy(
                src_ref=out_ref.at[b, r * RC:(r + 1) * RC],
                dst_ref=recv_ref.at[idx],
                send_sem=send_sems.at[idx],
                recv_sem=recv_sems.at[idx],
                device_id=(partner,),
                device_id_type=pl.DeviceIdType.MESH,
            )

        order = [(0, 0), (1, 0), (0, 1), (1, 1)]
        stage0 = {}
        for i, (b, r) in enumerate(order):
            compute_chunk(b, r)
            if i == 0:
                pl.semaphore_wait(barrier, 2)
            rdma = exchange(0, b, r, partner1 if b == 0 else partner2)
            rdma.start()
            stage0[(b, r)] = rdma

        stage1 = {}
        for b, r in order:
            stage0[(b, r)].wait()
            rows = slice(r * RC, (r + 1) * RC)
            out_ref[b, rows] = out_ref[b, rows] + recv_ref[b * 2 + r]
            rdma = exchange(1, b, r, partner2 if b == 0 else partner1)
            rdma.start()
            stage1[(b, r)] = rdma

        for b, r in order:
            stage1[(b, r)].wait()
            rows = slice(r * RC, (r + 1) * RC)
            out_ref[b, rows] = out_ref[b, rows] + recv_ref[4 + b * 2 + r]

    return pl.pallas_call(
        body,
        out_shape=jax.ShapeDtypeStruct((B, SQ, DM), jnp.float32),
        in_specs=[pl.BlockSpec(memory_space=pltpu.VMEM)] * 5,
        out_specs=pl.BlockSpec(memory_space=pltpu.VMEM),
        scratch_shapes=[
            pltpu.VMEM((8, SQ // 2, DM), jnp.float32),
            pltpu.SemaphoreType.DMA((8,)),
            pltpu.SemaphoreType.DMA((8,)),
        ],
        compiler_params=pltpu.CompilerParams(collective_id=0),
    )(x, Wq, K_loc, V_loc, Wo)
